# baseline (speedup 1.0000x reference)
import functools
import math

import jax
import jax.numpy as jnp
from jax import lax
from jax.experimental import pallas as pl
from jax.experimental.pallas import tpu as pltpu
from jax.experimental.pallas import tpu_sc as plsc

VOCAB = 1000000
D_MODEL = 64
SCALE = math.sqrt(D_MODEL)

NUM_CORES = 2
NUM_SUBCORES = 16
NUM_WORKERS = NUM_CORES * NUM_SUBCORES
LANES = 16

A_BLK = 256
A_FULL = VOCAB // A_BLK
A_UNIFORM = 122
A_TAIL = VOCAB - A_FULL * A_BLK


def _transpose_block(tbuf, obuf, iota64, n_chunks, scale):
    def chunk(c, carry):
        for d in range(D_MODEL):
            v = tbuf[d, pl.ds(c * LANES, LANES)] * scale
            plsc.store_scatter(obuf, [iota64 + (c * (LANES * D_MODEL) + d)], v)
        return carry

    lax.fori_loop(0, n_chunks, chunk, 0)


def _make_stage_a():
    mesh = plsc.VectorSubcoreMesh(core_axis_name="c", subcore_axis_name="s")

    @functools.partial(
        pl.kernel,
        mesh=mesh,
        out_type=jax.ShapeDtypeStruct((VOCAB * D_MODEL,), jnp.float32),
        scratch_types=[
            [pltpu.VMEM((D_MODEL, A_BLK), jnp.float32) for _ in range(2)],
            [pltpu.VMEM((A_BLK * D_MODEL,), jnp.float32) for _ in range(2)],
            pltpu.VMEM((A_TAIL, D_MODEL), jnp.float32),
            [pltpu.SemaphoreType.DMA for _ in range(2)],
            [pltpu.SemaphoreType.DMA for _ in range(2)],
        ],
        compiler_params=pltpu.CompilerParams(
            use_tc_tiling_on_sc=True,
            needs_layout_passes=False,
            disable_bounds_checks=True
        ),
    )
    def stage_a(tt_hbm, tail_hbm, out_hbm, tbufs, obufs, tailbuf, isems, osems):
        wid = lax.axis_index("s") * NUM_CORES + lax.axis_index("c")
        iota64 = lax.iota(jnp.int32, LANES) * D_MODEL

        def bid_of(k):
            return k * NUM_WORKERS + wid

        def issue_in(k, b):
            pltpu.async_copy(
                tt_hbm.at[:, pl.ds(bid_of(k) * A_BLK, A_BLK)], tbufs[b], isems[b]
            )

        def wait_in(b):
            pltpu.make_async_copy(
                tt_hbm.at[:, pl.ds(0, A_BLK)], tbufs[b], isems[b]
            ).wait()

        def issue_out(k, b):
            pltpu.async_copy(
                obufs[b],
                out_hbm.at[pl.ds(bid_of(k) * A_BLK * D_MODEL, A_BLK * D_MODEL)],
                osems[b],
            )

        def wait_out(b):
            pltpu.make_async_copy(
                obufs[b],
                out_hbm.at[pl.ds(0, A_BLK * D_MODEL)],
                osems[b],
            ).wait()

        issue_in(0, 0)

        def step(k, b):
            @pl.when(k + 1 < A_UNIFORM)
            def _():
                issue_in(k + 1, (b + 1) % 2)

            wait_in(b)

            @pl.when(k >= 2)
            def _():
                wait_out(b)

            _transpose_block(tbufs[b], obufs[b], iota64, A_BLK // LANES, SCALE)
            issue_out(k, b)

        def outer(t, carry):
            step(t * 2, 0)
            step(t * 2 + 1, 1)
            return carry

        lax.fori_loop(0, A_UNIFORM // 2, outer, 0)
        wait_out(0)
        wait_out(1)

        @pl.when(wid < A_FULL - A_UNIFORM * NUM_WORKERS)
        def _():
            blk = A_UNIFORM * NUM_WORKERS + wid
            pltpu.sync_copy(tt_hbm.at[:, pl.ds(blk * A_BLK, A_BLK)], tbufs[0])
            _transpose_block(tbufs[0], obufs[0], iota64, A_BLK // LANES, SCALE)
            pltpu.sync_copy(
                obufs[0],
                out_hbm.at[pl.ds(blk * A_BLK * D_MODEL, A_BLK * D_MODEL)],
            )

        @pl.when(wid == 2)
        def _():
            base = A_FULL * A_BLK
            pltpu.sync_copy(tail_hbm, tailbuf)

            def row(r, carry):
                for c in range(D_MODEL // LANES):
                    sl = pl.ds(c * LANES, LANES)
                    obufs[0][pl.ds(r * D_MODEL + c * LANES, LANES)] = (
                        tailbuf[r, sl] * SCALE
                    )
                return carry

            lax.fori_loop(0, A_TAIL, row, 0)
            pltpu.sync_copy(
                obufs[0].at[pl.ds(0, A_TAIL * D_MODEL)],
                out_hbm.at[pl.ds(base * D_MODEL, A_TAIL * D_MODEL)],
            )

    return stage_a


B_TOK = 128
N_SEQ = 200
B_NBUF = 4


def _make_stage_b():
    mesh = plsc.VectorSubcoreMesh(core_axis_name="c", subcore_axis_name="s")
    ids_per_w = B_TOK * N_SEQ

    @functools.partial(
        pl.kernel,
        mesh=mesh,
        out_type=jax.ShapeDtypeStruct(
            (N_SEQ, D_MODEL // 8, NUM_WORKERS, 8, B_TOK), jnp.float32
        ),
        scratch_types=[
            pltpu.VMEM((ids_per_w,), jnp.int32),
            pltpu.VMEM((ids_per_w,), jnp.int32),
            [pltpu.VMEM((B_TOK, D_MODEL), jnp.float32) for _ in range(B_NBUF)],
            [pltpu.VMEM((D_MODEL // 8, 8, B_TOK), jnp.float32) for _ in range(B_NBUF)],
            [pltpu.SemaphoreType.DMA for _ in range(B_NBUF)],
            [pltpu.SemaphoreType.DMA for _ in range(B_NBUF)],
        ],
        compiler_params=pltpu.CompilerParams(
            use_tc_tiling_on_sc=False,
            needs_layout_passes=False,
            disable_bounds_checks=True
        ),
    )
    def stage_b(ids_hbm, tab_hbm, out_hbm, ids_v, idst_v, gbufs, obufs, gsems, osems):
        wid = lax.axis_index("s") * NUM_CORES + lax.axis_index("c")
        iota = lax.iota(jnp.int32, LANES)
        iota_nseq = iota * N_SEQ
        td_base = iota // 8
        d8 = iota % 8

        pltpu.sync_copy(ids_hbm.at[pl.ds(wid * ids_per_w, ids_per_w)], ids_v)

        def ids_t(s, carry):
            for t0 in range(0, B_TOK, LANES):
                v = plsc.load_gather(ids_v, [iota_nseq + (t0 * N_SEQ) + s])
                idst_v[pl.ds(s * B_TOK + t0, LANES)] = v
            return carry

        lax.fori_loop(0, N_SEQ, ids_t, 0)

        def issue_gather(s, b):
            pltpu.async_copy(
                tab_hbm.at[idst_v.at[pl.ds(s * B_TOK, B_TOK)]], gbufs[b], gsems[b]
            )

        def wait_gather(b):
            pltpu.make_async_copy(
                tab_hbm.at[idst_v.at[pl.ds(0, B_TOK)]], gbufs[b], gsems[b]
            ).wait()

        def issue_out(s, b):
            pltpu.async_copy(obufs[b], out_hbm.at[s, :, wid], osems[b])

        def wait_out(b):
            pltpu.make_async_copy(obufs[b], out_hbm.at[0, :, wid], osems[b]).wait()

        for b in range(B_NBUF - 1):
            issue_gather(b, b)

        def step(s, b):
            @pl.when(s + B_NBUF - 1 < N_SEQ)
            def _():
                issue_gather(s + B_NBUF - 1, (b + B_NBUF - 1) % B_NBUF)

            wait_gather(b)

            @pl.when(s >= B_NBUF)
            def _():
                wait_out(b)

            def tok(t, carry):
                for d0 in range(0, D_MODEL, LANES):
                    v = gbufs[b][t, pl.ds(d0, LANES)]
                    plsc.store_scatter(
                        obufs[b],
                        [td_base + (d0 // 8), d8, jnp.full((LANES,), t, jnp.int32)],
                        v,
                    )
                return carry

            lax.fori_loop(0, B_TOK, tok, 0, unroll=4)
            issue_out(s, b)

        def outer(t, carry):
            for b in range(B_NBUF):
                step(t * B_NBUF + b, b)
            return carry

        lax.fori_loop(0, N_SEQ // B_NBUF, outer, 0)
        for b in range(B_NBUF):
            wait_out(b)

    return stage_b


def kernel(input_ids, table):
    flat_ids = input_ids.reshape(-1).astype(jnp.int32)
    tail = table[A_FULL * A_BLK:, :]
    table_scaled = _make_stage_a()(table.T, tail).reshape(VOCAB, D_MODEL)
    out5 = _make_stage_b()(flat_ids, table_scaled)
    return jnp.transpose(out5, (2, 4, 0, 1, 3)).reshape(*input_ids.shape, D_MODEL)


# speedup vs baseline: 1.5196x; 1.5196x over previous
import functools
import math

import jax
import jax.numpy as jnp
from jax import lax
from jax.experimental import pallas as pl
from jax.experimental.pallas import tpu as pltpu
from jax.experimental.pallas import tpu_sc as plsc

VOCAB = 1000000
D_MODEL = 64
SCALE = math.sqrt(D_MODEL)

NUM_CORES = 2
NUM_SUBCORES = 16
NUM_WORKERS = NUM_CORES * NUM_SUBCORES
LANES = 16

BATCH = 4096
N_SEQ = 200
TOK_PER_W = BATCH // NUM_WORKERS
NBUF = 4


def _make_kernel():
    mesh = plsc.VectorSubcoreMesh(core_axis_name="c", subcore_axis_name="s")
    ids_per_w = TOK_PER_W * N_SEQ

    @functools.partial(
        pl.kernel,
        mesh=mesh,
        out_type=jax.ShapeDtypeStruct((BATCH, N_SEQ, D_MODEL), jnp.float32),
        scratch_types=[
            pltpu.VMEM((ids_per_w,), jnp.int32),
            [pltpu.VMEM((N_SEQ, D_MODEL), jnp.float32) for _ in range(NBUF)],
            [pltpu.VMEM((N_SEQ, D_MODEL), jnp.float32) for _ in range(NBUF)],
            [pltpu.SemaphoreType.DMA for _ in range(NBUF)],
            [pltpu.SemaphoreType.DMA for _ in range(NBUF)],
        ],
        compiler_params=pltpu.CompilerParams(
            use_tc_tiling_on_sc=False,
            needs_layout_passes=False,
            disable_bounds_checks=True,
        ),
    )
    def gather_scale(ids_hbm, tab_hbm, out_hbm, ids_v, gbufs, obufs, gsems, osems):
        wid = lax.axis_index("s") * NUM_CORES + lax.axis_index("c")
        tok0 = wid * TOK_PER_W

        pltpu.sync_copy(ids_hbm.at[pl.ds(wid * ids_per_w, ids_per_w)], ids_v)

        def issue_gather(t, b):
            pltpu.async_copy(
                tab_hbm.at[ids_v.at[pl.ds(t * N_SEQ, N_SEQ)]], gbufs[b], gsems[b]
            )

        def wait_gather(b):
            pltpu.make_async_copy(
                tab_hbm.at[ids_v.at[pl.ds(0, N_SEQ)]], gbufs[b], gsems[b]
            ).wait()

        def issue_out(t, b):
            pltpu.async_copy(obufs[b], out_hbm.at[tok0 + t], osems[b])

        def wait_out(b):
            pltpu.make_async_copy(obufs[b], out_hbm.at[0], osems[b]).wait()

        for b in range(NBUF - 1):
            issue_gather(b, b)

        def step(t, b):
            @pl.when(t + NBUF - 1 < TOK_PER_W)
            def _():
                issue_gather(t + NBUF - 1, (b + NBUF - 1) % NBUF)

            wait_gather(b)

            @pl.when(t >= NBUF)
            def _():
                wait_out(b)

            def row(s, carry):
                for d0 in range(0, D_MODEL, LANES):
                    sl = pl.ds(d0, LANES)
                    obufs[b][s, sl] = gbufs[b][s, sl] * SCALE
                return carry

            lax.fori_loop(0, N_SEQ, row, 0, unroll=4)
            issue_out(t, b)

        def outer(q, carry):
            for b in range(NBUF):
                step(q * NBUF + b, b)
            return carry

        lax.fori_loop(0, TOK_PER_W // NBUF, outer, 0)
        for b in range(NBUF):
            wait_out(b)

    return gather_scale


def kernel(input_ids, table):
    flat_ids = input_ids.reshape(-1).astype(jnp.int32)
    return _make_kernel()(flat_ids, table)


# speedup vs baseline: 1.5315x; 1.0078x over previous
import functools
import math

import jax
import jax.numpy as jnp
from jax import lax
from jax.experimental import pallas as pl
from jax.experimental.pallas import tpu as pltpu
from jax.experimental.pallas import tpu_sc as plsc

VOCAB = 1000000
D_MODEL = 64
SCALE = math.sqrt(D_MODEL)

NUM_CORES = 2
NUM_SUBCORES = 16
NUM_WORKERS = NUM_CORES * NUM_SUBCORES
LANES = 16

BATCH = 4096
N_SEQ = 200
TOK_PER_W = BATCH // NUM_WORKERS
NBUF = 4


def _make_kernel():
    mesh = plsc.VectorSubcoreMesh(core_axis_name="c", subcore_axis_name="s")
    ids_per_w = TOK_PER_W * N_SEQ

    @functools.partial(
        pl.kernel,
        mesh=mesh,
        out_type=jax.ShapeDtypeStruct((BATCH, N_SEQ, D_MODEL), jnp.float32),
        scratch_types=[
            pltpu.VMEM((ids_per_w,), jnp.int32),
            [pltpu.VMEM((N_SEQ, D_MODEL), jnp.float32) for _ in range(NBUF)],
            [pltpu.SemaphoreType.DMA for _ in range(NBUF)],
            [pltpu.SemaphoreType.DMA for _ in range(NBUF)],
        ],
        compiler_params=pltpu.CompilerParams(
            use_tc_tiling_on_sc=False,
            needs_layout_passes=False,
            disable_bounds_checks=True,
        ),
    )
    def gather_scale(ids_hbm, tab_hbm, out_hbm, ids_v, gbufs, gsems, osems):
        wid = lax.axis_index("s") * NUM_CORES + lax.axis_index("c")
        tok0 = wid * TOK_PER_W

        pltpu.sync_copy(ids_hbm.at[pl.ds(wid * ids_per_w, ids_per_w)], ids_v)

        def issue_gather(t, b):
            pltpu.async_copy(
                tab_hbm.at[ids_v.at[pl.ds(t * N_SEQ, N_SEQ)]], gbufs[b], gsems[b]
            )

        def wait_gather(b):
            pltpu.make_async_copy(
                tab_hbm.at[ids_v.at[pl.ds(0, N_SEQ)]], gbufs[b], gsems[b]
            ).wait()

        def issue_out(t, b):
            pltpu.async_copy(gbufs[b], out_hbm.at[tok0 + t], osems[b])

        def wait_out(b):
            pltpu.make_async_copy(gbufs[b], out_hbm.at[0], osems[b]).wait()

        for b in range(NBUF - 1):
            issue_gather(b, b)

        def step(t, b):
            @pl.when(t + NBUF - 1 < TOK_PER_W)
            def _():
                nb = (b + NBUF - 1) % NBUF

                @pl.when(t >= 1)
                def _():
                    wait_out(nb)

                issue_gather(t + NBUF - 1, nb)

            wait_gather(b)
            issue_out(t, b)

        def outer(q, carry):
            for b in range(NBUF):
                step(q * NBUF + b, b)
            return carry

        lax.fori_loop(0, TOK_PER_W // NBUF, outer, 0)
        for b in range(NBUF):
            wait_out(b)

    return gather_scale


def kernel(input_ids, table):
    flat_ids = input_ids.reshape(-1).astype(jnp.int32)
    return _make_kernel()(flat_ids, table * SCALE)


# speedup vs baseline: 1.5316x; 1.0001x over previous
import functools
import math

import jax
import jax.numpy as jnp
from jax import lax
from jax.experimental import pallas as pl
from jax.experimental.pallas import tpu as pltpu
from jax.experimental.pallas import tpu_sc as plsc

VOCAB = 1000000
D_MODEL = 64
SCALE = math.sqrt(D_MODEL)

NUM_CORES = 2
NUM_SUBCORES = 16
NUM_WORKERS = NUM_CORES * NUM_SUBCORES
LANES = 16

BATCH = 4096
N_SEQ = 200
TOK_PER_W = BATCH // NUM_WORKERS
NBUF = 8


def _make_kernel():
    mesh = plsc.VectorSubcoreMesh(core_axis_name="c", subcore_axis_name="s")
    ids_per_w = TOK_PER_W * N_SEQ

    @functools.partial(
        pl.kernel,
        mesh=mesh,
        out_type=jax.ShapeDtypeStruct((BATCH, N_SEQ, D_MODEL), jnp.float32),
        scratch_types=[
            pltpu.VMEM((ids_per_w,), jnp.int32),
            [pltpu.VMEM((N_SEQ, D_MODEL), jnp.float32) for _ in range(NBUF)],
            [pltpu.SemaphoreType.DMA for _ in range(NBUF)],
            [pltpu.SemaphoreType.DMA for _ in range(NBUF)],
        ],
        compiler_params=pltpu.CompilerParams(
            use_tc_tiling_on_sc=False,
            needs_layout_passes=False,
            disable_bounds_checks=True,
        ),
    )
    def gather_scale(ids_hbm, tab_hbm, out_hbm, ids_v, gbufs, gsems, osems):
        wid = lax.axis_index("s") * NUM_CORES + lax.axis_index("c")
        tok0 = wid * TOK_PER_W

        pltpu.sync_copy(ids_hbm.at[pl.ds(wid * ids_per_w, ids_per_w)], ids_v)

        def issue_gather(t, b):
            pltpu.async_copy(
                tab_hbm.at[ids_v.at[pl.ds(t * N_SEQ, N_SEQ)]], gbufs[b], gsems[b]
            )

        def wait_gather(b):
            pltpu.make_async_copy(
                tab_hbm.at[ids_v.at[pl.ds(0, N_SEQ)]], gbufs[b], gsems[b]
            ).wait()

        def issue_out(t, b):
            pltpu.async_copy(gbufs[b], out_hbm.at[tok0 + t], osems[b])

        def wait_out(b):
            pltpu.make_async_copy(gbufs[b], out_hbm.at[0], osems[b]).wait()

        for b in range(NBUF - 1):
            issue_gather(b, b)

        def step(t, b):
            @pl.when(t + NBUF - 1 < TOK_PER_W)
            def _():
                nb = (b + NBUF - 1) % NBUF

                @pl.when(t >= 1)
                def _():
                    wait_out(nb)

                issue_gather(t + NBUF - 1, nb)

            wait_gather(b)
            issue_out(t, b)

        def outer(q, carry):
            for b in range(NBUF):
                step(q * NBUF + b, b)
            return carry

        lax.fori_loop(0, TOK_PER_W // NBUF, outer, 0)
        for b in range(NBUF):
            wait_out(b)

    return gather_scale


def kernel(input_ids, table):
    flat_ids = input_ids.reshape(-1).astype(jnp.int32)
    return _make_kernel()(flat_ids, table * SCALE)
